# baseline (speedup 1.0000x reference)
import functools

import jax
import jax.numpy as jnp
from jax import lax
from jax.experimental import pallas as pl
from jax.experimental.pallas import tpu as pltpu
from jax.experimental.pallas import tpu_sc as plsc

NUM_EMB = 1_000_000
DIM = 16
BATCH = 16384
HIST = 20
B_TOTAL = BATCH * HIST
NUM_WORKERS = 32
B_PER_W = B_TOTAL // NUM_WORKERS
CHUNK = 2048
N_CHUNKS = B_PER_W // CHUNK

_mesh = plsc.VectorSubcoreMesh(core_axis_name="c", subcore_axis_name="s")


@functools.partial(
    pl.kernel,
    mesh=_mesh,
    out_type=jax.ShapeDtypeStruct((B_TOTAL, DIM), jnp.float32),
    scratch_types=[
        pltpu.VMEM((B_PER_W,), jnp.int32),
        pltpu.VMEM((2, CHUNK, DIM), jnp.float32),
        pltpu.SemaphoreType.DMA,
        pltpu.SemaphoreType.DMA,
    ],
    compiler_params=pltpu.CompilerParams(use_tc_tiling_on_sc=False),
)
def _emb_lookup(idx_hbm, table_hbm, out_hbm, idx_v, rows_v, gsem, ssem):
    wid = lax.axis_index("s") * 2 + lax.axis_index("c")
    base = wid * B_PER_W
    pltpu.sync_copy(idx_hbm.at[pl.ds(base, B_PER_W)], idx_v)

    gathers = [None, None]
    outs = [None, None]
    gathers[0] = pltpu.async_copy(
        table_hbm.at[idx_v.at[pl.ds(0, CHUNK)]], rows_v.at[0], gsem
    )
    for j in range(N_CHUNKS):
        buf = j % 2
        gathers[buf].wait()
        if j + 1 < N_CHUNKS:
            nbuf = (j + 1) % 2
            if outs[nbuf] is not None:
                outs[nbuf].wait()
            gathers[nbuf] = pltpu.async_copy(
                table_hbm.at[idx_v.at[pl.ds((j + 1) * CHUNK, CHUNK)]],
                rows_v.at[nbuf],
                gsem,
            )
        outs[buf] = pltpu.async_copy(
            rows_v.at[buf], out_hbm.at[pl.ds(base + j * CHUNK, CHUNK)], ssem
        )
    for o in outs:
        if o is not None:
            o.wait()


def kernel(x, emb_weight):
    idx = x.astype(jnp.int32).reshape(B_TOTAL)
    out = _emb_lookup(idx, emb_weight)
    return out.reshape(BATCH, HIST, DIM)


# speedup vs baseline: 6.5174x; 6.5174x over previous
import functools

import jax
import jax.numpy as jnp
from jax import lax
from jax.experimental import pallas as pl
from jax.experimental.pallas import tpu as pltpu
from jax.experimental.pallas import tpu_sc as plsc

NUM_EMB = 1_000_000
DIM = 16
BATCH = 16384
HIST = 20

NUM_CORES = 2
NUM_SUBCORES = 16
D_PER_CORE = DIM // NUM_CORES
B_BLOCK = BATCH // NUM_SUBCORES
STAGE_CHUNK = 124928
STAGE_TAIL_OFF = 8 * STAGE_CHUNK
STAGE_TAIL = NUM_EMB - STAGE_TAIL_OFF

_mesh = plsc.VectorSubcoreMesh(core_axis_name="c", subcore_axis_name="s")


@functools.partial(
    pl.kernel,
    mesh=_mesh,
    out_type=jax.ShapeDtypeStruct((HIST, DIM, BATCH), jnp.float32),
    scratch_types=[
        pltpu.VMEM((HIST * B_BLOCK,), jnp.int32),
        pltpu.VMEM((1, B_BLOCK), jnp.float32),
        pltpu.VMEM((1, B_BLOCK), jnp.float32),
        pltpu.VMEM_SHARED((1, NUM_EMB), jnp.float32),
        pltpu.SemaphoreType.DMA,
        pltpu.SemaphoreType.DMA,
        pltpu.SemaphoreType.DMA,
        pltpu.SemaphoreType.DMA,
        pltpu.SemaphoreType.DMA,
    ],
)
def _emb_lookup(
    xt_hbm, tt_hbm, out_hbm,
    idx_v, rows0, rows1, drow0,
    ssem, gsem0, gsem1, wsem0, wsem1,
):
    cid = lax.axis_index("c")
    sid = lax.axis_index("s")
    dbase = cid * D_PER_CORE
    b0 = sid * B_BLOCK
    rows = [rows0, rows1]
    gsem = [gsem0, gsem1]
    wsem = [wsem0, wsem1]

    for h in range(HIST):
        pltpu.sync_copy(
            xt_hbm.at[h, pl.ds(b0, B_BLOCK)],
            idx_v.at[pl.ds(h * B_BLOCK, B_BLOCK)],
        )

    def _stage_args(d):
        grp = (d % 2) * 8
        i = sid - grp
        off = pl.multiple_of(i * STAGE_CHUNK, 128)
        src = tt_hbm.at[pl.ds(dbase + d, 1), pl.ds(off, STAGE_CHUNK)]
        dst = drow0.at[:, pl.ds(off, STAGE_CHUNK)]
        tsrc = tt_hbm.at[pl.ds(dbase + d, 1), pl.ds(STAGE_TAIL_OFF, STAGE_TAIL)]
        tdst = drow0.at[:, pl.ds(STAGE_TAIL_OFF, STAGE_TAIL)]
        return grp, src, dst, tsrc, tdst

    def stage_start(d):
        grp, src, dst, tsrc, tdst = _stage_args(d)

        @pl.when((sid >= grp) & (sid < grp + 8))
        def _():
            pltpu.async_copy(src, dst, ssem)

        @pl.when(sid == grp)
        def _():
            pltpu.async_copy(tsrc, tdst, ssem)

    def stage_wait(d):
        grp, src, dst, tsrc, tdst = _stage_args(d)

        @pl.when((sid >= grp) & (sid < grp + 8))
        def _():
            pltpu.make_async_copy(src, dst, ssem).wait()

        @pl.when(sid == grp)
        def _():
            pltpu.make_async_copy(tsrc, tdst, ssem).wait()

    for d in range(D_PER_CORE):
        stage_start(d)
        stage_wait(d)
        plsc.subcore_barrier()

        gathers = [None, None]
        writes = [None, None]
        for h in range(HIST):
            rb = h % 2
            if writes[rb] is not None:
                writes[rb].wait()
            gathers[rb] = pltpu.async_copy(
                drow0.at[0].at[idx_v.at[pl.ds(h * B_BLOCK, B_BLOCK)]],
                rows[rb].at[0],
                gsem[rb],
            )
            if h >= 1:
                pb = 1 - rb
                gathers[pb].wait()
                writes[pb] = pltpu.async_copy(
                    rows[pb],
                    out_hbm.at[h - 1, pl.ds(dbase + d, 1), pl.ds(b0, B_BLOCK)],
                    wsem[pb],
                )
        last = (HIST - 1) % 2
        gathers[last].wait()
        writes[last] = pltpu.async_copy(
            rows[last],
            out_hbm.at[HIST - 1, pl.ds(dbase + d, 1), pl.ds(b0, B_BLOCK)],
            wsem[last],
        )
        writes[0].wait()
        writes[1].wait()

        plsc.subcore_barrier()


def kernel(x, emb_weight):
    out = _emb_lookup(x.T.astype(jnp.int32), emb_weight.T)
    return out.transpose(2, 0, 1)


# speedup vs baseline: 8.8123x; 1.3521x over previous
import functools

import jax
import jax.numpy as jnp
from jax import lax
from jax.experimental import pallas as pl
from jax.experimental.pallas import tpu as pltpu
from jax.experimental.pallas import tpu_sc as plsc

NUM_EMB = 1_000_000
DIM = 16
BATCH = 16384
HIST = 20

NUM_CORES = 2
NUM_SUBCORES = 16
D_PER_CORE = DIM // NUM_CORES
B_BLOCK = BATCH // NUM_SUBCORES
STAGE_CHUNK = 124928
STAGE_TAIL_OFF = 8 * STAGE_CHUNK
STAGE_TAIL = NUM_EMB - STAGE_TAIL_OFF

_mesh = plsc.VectorSubcoreMesh(core_axis_name="c", subcore_axis_name="s")


@functools.partial(
    pl.kernel,
    mesh=_mesh,
    out_type=jax.ShapeDtypeStruct((HIST, DIM, BATCH), jnp.float32),
    scratch_types=[
        pltpu.VMEM((B_BLOCK,), jnp.int32),
        pltpu.VMEM((B_BLOCK,), jnp.int32),
        pltpu.VMEM((B_BLOCK,), jnp.int32),
        pltpu.VMEM((1, B_BLOCK), jnp.float32),
        pltpu.VMEM((1, B_BLOCK), jnp.float32),
        pltpu.VMEM_SHARED((1, NUM_EMB), jnp.float32),
        pltpu.VMEM_SHARED((1, NUM_EMB), jnp.float32),
        pltpu.SemaphoreType.DMA,
        pltpu.SemaphoreType.DMA,
        pltpu.SemaphoreType.DMA,
        pltpu.SemaphoreType.DMA,
        pltpu.SemaphoreType.DMA,
        pltpu.SemaphoreType.DMA,
        pltpu.SemaphoreType.DMA,
        pltpu.SemaphoreType.DMA,
    ],
)
def _emb_lookup(
    xt_hbm, tt_hbm, out_hbm,
    idx0, idx1, idx2, rows0, rows1, drow0, drow1,
    ssem, isem0, isem1, isem2, gsem0, gsem1, wsem0, wsem1,
):
    cid = lax.axis_index("c")
    sid = lax.axis_index("s")
    dbase = cid * D_PER_CORE
    b0 = sid * B_BLOCK
    idxs = [idx0, idx1, idx2]
    isem = [isem0, isem1, isem2]
    rows = [rows0, rows1]
    gsem = [gsem0, gsem1]
    wsem = [wsem0, wsem1]
    drow = [drow0, drow1]
    T = D_PER_CORE * HIST

    def idx_start(t):
        h = t % HIST
        pltpu.async_copy(
            xt_hbm.at[h, pl.ds(b0, B_BLOCK)], idxs[t % 3], isem[t % 3]
        )

    def idx_wait(t):
        h = t % HIST
        pltpu.make_async_copy(
            xt_hbm.at[h, pl.ds(b0, B_BLOCK)], idxs[t % 3], isem[t % 3]
        ).wait()

    def _stage_args(d):
        grp = (d % 2) * 8
        i = sid - grp
        off = pl.multiple_of(i * STAGE_CHUNK, 128)
        src = tt_hbm.at[pl.ds(dbase + d, 1), pl.ds(off, STAGE_CHUNK)]
        dst = drow[d % 2].at[:, pl.ds(off, STAGE_CHUNK)]
        tsrc = tt_hbm.at[pl.ds(dbase + d, 1), pl.ds(STAGE_TAIL_OFF, STAGE_TAIL)]
        tdst = drow[d % 2].at[:, pl.ds(STAGE_TAIL_OFF, STAGE_TAIL)]
        return grp, src, dst, tsrc, tdst

    def stage_start(d):
        grp, src, dst, tsrc, tdst = _stage_args(d)

        @pl.when((sid >= grp) & (sid < grp + 8))
        def _():
            pltpu.async_copy(src, dst, ssem)

        @pl.when(sid == grp)
        def _():
            pltpu.async_copy(tsrc, tdst, ssem)

    def stage_wait(d):
        grp, src, dst, tsrc, tdst = _stage_args(d)

        @pl.when((sid >= grp) & (sid < grp + 8))
        def _():
            pltpu.make_async_copy(src, dst, ssem).wait()

        @pl.when(sid == grp)
        def _():
            pltpu.make_async_copy(tsrc, tdst, ssem).wait()

    stage_start(0)
    idx_start(0)
    idx_start(1)
    idx_start(2)
    stage_wait(0)
    plsc.subcore_barrier()

    writes = [None, None]
    gathers = [None, None]
    for d in range(D_PER_CORE):
        buf = d % 2
        if d + 1 < D_PER_CORE:
            stage_start(d + 1)

        for h in range(HIST):
            t = d * HIST + h
            rb = t % 2
            idx_wait(t)
            if writes[rb] is not None:
                writes[rb].wait()
                writes[rb] = None
            gathers[rb] = pltpu.async_copy(
                drow[buf].at[0].at[idxs[t % 3]],
                rows[rb].at[0],
                gsem[rb],
            )
            if h >= 1:
                pb = 1 - rb
                gathers[pb].wait()
                writes[pb] = pltpu.async_copy(
                    rows[pb],
                    out_hbm.at[h - 1, pl.ds(dbase + d, 1), pl.ds(b0, B_BLOCK)],
                    wsem[pb],
                )
            if 1 <= t < T - 2:
                idx_start(t + 2)
        tl = (HIST - 1) % 2
        gathers[tl].wait()
        writes[tl] = pltpu.async_copy(
            rows[tl],
            out_hbm.at[HIST - 1, pl.ds(dbase + d, 1), pl.ds(b0, B_BLOCK)],
            wsem[tl],
        )

        if d + 1 < D_PER_CORE:
            stage_wait(d + 1)
        plsc.subcore_barrier()

    for rb in range(2):
        if writes[rb] is not None:
            writes[rb].wait()


def kernel(x, emb_weight):
    out = _emb_lookup(x.T.astype(jnp.int32), emb_weight.T)
    return out.transpose(2, 0, 1)
